# baseline (speedup 1.0000x reference)
import jax
import jax.numpy as jnp
from jax.experimental import pallas as pl
from jax.experimental.pallas import tpu as pltpu


_HB = 8
_DEPTH = 8


def _body(x_hbm, row_ref, col_ref, o_hbm, xb, ob, posb, in_sems, out_sems):
    nb = x_hbm.shape[0]
    h = x_hbm.shape[1]
    sub = x_hbm.shape[2]
    hb = posb.shape[0]
    n = (h // hb) * nb
    i = pl.program_id(0)
    hi = i // nb
    bi = i % nb
    slot = jax.lax.rem(i, _DEPTH)

    def read(step):
        s_hi = step // nb
        s_bi = step % nb
        s_slot = jax.lax.rem(step, _DEPTH)
        pltpu.make_async_copy(
            x_hbm.at[s_bi, pl.ds(s_hi * hb, hb)],
            xb.at[s_slot],
            in_sems.at[s_slot],
        ).start()

    @pl.when(i == 0)
    def _prologue():
        for d in range(_DEPTH):
            read(jnp.int32(d))

    pltpu.make_async_copy(
        x_hbm.at[bi, pl.ds(hi * hb, hb)], xb.at[slot], in_sems.at[slot]
    ).wait()

    @pl.when(bi == 0)
    def _pos():
        row24 = row_ref[pl.ds(hi * hb, hb), :, :]
        col = col_ref[...]
        rowpat = jnp.broadcast_to(row24[:, None, :, :], (hb, sub // 24, 24, 128))
        posb[...] = rowpat.reshape(hb, sub, 128) + col[None, :, :]

    @pl.when(i >= _DEPTH)
    def _drain_out():
        pltpu.make_async_copy(
            ob.at[slot], o_hbm.at[bi, pl.ds(hi * hb, hb)], out_sems.at[slot]
        ).wait()

    ob[slot] = xb[slot] + posb[...]

    pltpu.make_async_copy(
        ob.at[slot], o_hbm.at[bi, pl.ds(hi * hb, hb)], out_sems.at[slot]
    ).start()

    @pl.when(i + _DEPTH < n)
    def _next_read():
        read(i + _DEPTH)

    @pl.when(i == n - 1)
    def _epilogue():
        for d in range(_DEPTH):
            step = n - _DEPTH + d
            s_hi = step // nb
            s_bi = step % nb
            pltpu.make_async_copy(
                ob.at[d], o_hbm.at[s_bi, pl.ds(s_hi * hb, hb)], out_sems.at[d]
            ).wait()


def kernel(inputs, row_embeddings, col_embeddings):
    b, h, w, c = inputs.shape
    hb = _HB
    sub = (w * c) // 128
    n = (h // hb) * b
    x = inputs.reshape(b, h, sub, 128)
    rowcat = jnp.concatenate([row_embeddings] * 3, axis=1)
    pieces = [rowcat[:, (p * 128) % c:(p * 128) % c + 128] for p in range(3)]
    row24 = jnp.tile(jnp.stack(pieces, axis=1), (1, 8, 1))
    col2d = col_embeddings.reshape(sub, 128)
    out = pl.pallas_call(
        _body,
        grid=(n,),
        in_specs=[
            pl.BlockSpec(memory_space=pltpu.MemorySpace.HBM),
            pl.BlockSpec((h, 24, 128), lambda i: (0, 0, 0)),
            pl.BlockSpec((sub, 128), lambda i: (0, 0)),
        ],
        out_specs=pl.BlockSpec(memory_space=pltpu.MemorySpace.HBM),
        out_shape=jax.ShapeDtypeStruct((b, h, sub, 128), inputs.dtype),
        scratch_shapes=[
            pltpu.VMEM((_DEPTH, hb, sub, 128), inputs.dtype),
            pltpu.VMEM((_DEPTH, hb, sub, 128), inputs.dtype),
            pltpu.VMEM((hb, sub, 128), inputs.dtype),
            pltpu.SemaphoreType.DMA((_DEPTH,)),
            pltpu.SemaphoreType.DMA((_DEPTH,)),
        ],
        compiler_params=pltpu.CompilerParams(
            dimension_semantics=("arbitrary",),
        ),
    )(x, row24, col2d)
    return out.reshape(b, h, w, c)


# speedup vs baseline: 1.2844x; 1.2844x over previous
import jax
import jax.numpy as jnp
from jax.experimental import pallas as pl
from jax.experimental.pallas import tpu as pltpu


_HB = 4
_DEPTH = 2


def _body(x_hbm, row_ref, col_ref, o_hbm, xb, ob, in_sems, out_sems):
    h = x_hbm.shape[1]
    hb = _HB
    n = h // hb
    i = pl.program_id(0)
    slot = jax.lax.rem(i, _DEPTH)

    def read(step):
        s_slot = jax.lax.rem(step, _DEPTH)
        return pltpu.make_async_copy(
            x_hbm.at[:, pl.ds(step * hb, hb)],
            xb.at[s_slot],
            in_sems.at[s_slot],
        )

    def write(step):
        s_slot = jax.lax.rem(step, _DEPTH)
        return pltpu.make_async_copy(
            ob.at[s_slot],
            o_hbm.at[:, pl.ds(step * hb, hb)],
            out_sems.at[s_slot],
        )

    @pl.when(i == 0)
    def _prologue():
        for d in range(_DEPTH):
            read(jnp.int32(d)).start()

    read(i).wait()

    @pl.when(i >= _DEPTH)
    def _drain_out():
        write(i - _DEPTH).wait()

    row = row_ref[pl.ds(i * hb, hb), :]
    col = col_ref[...]
    pos = row[:, None, :] + col[None, :, :]
    ob[slot] = xb[slot] + pos[None, :, :, :]

    write(i).start()

    @pl.when(i + _DEPTH < n)
    def _next_read():
        read(i + _DEPTH).start()

    @pl.when(i == n - 1)
    def _epilogue():
        for d in range(_DEPTH):
            write(n - _DEPTH + jnp.int32(d)).wait()


def kernel(inputs, row_embeddings, col_embeddings):
    b, h, w, c = inputs.shape
    hb = _HB
    n = h // hb
    return pl.pallas_call(
        _body,
        grid=(n,),
        in_specs=[
            pl.BlockSpec(memory_space=pltpu.MemorySpace.HBM),
            pl.BlockSpec((h, c), lambda i: (0, 0)),
            pl.BlockSpec((w, c), lambda i: (0, 0)),
        ],
        out_specs=pl.BlockSpec(memory_space=pltpu.MemorySpace.HBM),
        out_shape=jax.ShapeDtypeStruct((b, h, w, c), inputs.dtype),
        scratch_shapes=[
            pltpu.VMEM((_DEPTH, b, hb, w, c), inputs.dtype),
            pltpu.VMEM((_DEPTH, b, hb, w, c), inputs.dtype),
            pltpu.SemaphoreType.DMA((_DEPTH,)),
            pltpu.SemaphoreType.DMA((_DEPTH,)),
        ],
        compiler_params=pltpu.CompilerParams(
            dimension_semantics=("arbitrary",),
        ),
    )(inputs, row_embeddings, col_embeddings)


# speedup vs baseline: 1.7866x; 1.3910x over previous
import jax
import jax.numpy as jnp
from jax.experimental import pallas as pl
from jax.experimental.pallas import tpu as pltpu
from jax.experimental.pallas import tpu_sc as plsc


_WB = 64
_LANES = 16


def kernel(inputs, row_embeddings, col_embeddings):
    b, h, w, c = inputs.shape
    wb = _WB
    mesh = plsc.VectorSubcoreMesh(core_axis_name="core", subcore_axis_name="subcore")

    @pl.kernel(
        out_type=jax.ShapeDtypeStruct((b, h, w, c), inputs.dtype),
        mesh=mesh,
        scratch_types=[],
    )
    def sc_kernel(x_hbm, row_hbm, col_hbm, o_hbm):
        def body(x_vmem, row_vmem, col_vmem, o_vmem):
            @pl.loop(0, wb)
            def _(wr):
                @pl.loop(0, c, step=_LANES)
                def _(cc):
                    rv = row_vmem.at[0, pl.ds(cc, _LANES)][...]
                    cv = col_vmem.at[wr, pl.ds(cc, _LANES)][...]
                    xv = x_vmem.at[0, 0, wr, pl.ds(cc, _LANES)][...]
                    o_vmem.at[0, 0, wr, pl.ds(cc, _LANES)][...] = xv + rv + cv

        pltpu.emit_pipeline(
            body,
            grid=(b, h, w // wb),
            in_specs=[
                pl.BlockSpec((1, 1, wb, c), index_map=lambda bi, hi, wi: (bi, hi, wi, 0)),
                pl.BlockSpec((1, c), index_map=lambda bi, hi, wi: (hi, 0)),
                pl.BlockSpec((wb, c), index_map=lambda bi, hi, wi: (wi, 0)),
            ],
            out_specs=[
                pl.BlockSpec((1, 1, wb, c), index_map=lambda bi, hi, wi: (bi, hi, wi, 0)),
            ],
            core_axis_name=("core", "subcore"),
            dimension_semantics=(pltpu.PARALLEL, pltpu.PARALLEL, pltpu.PARALLEL),
        )(x_hbm, row_hbm, col_hbm, o_hbm)

    return sc_kernel(inputs, row_embeddings, col_embeddings)
